# baseline (speedup 1.0000x reference)
import functools

import jax
import jax.numpy as jnp
from jax import lax
from jax.experimental import pallas as pl
from jax.experimental.pallas import tpu as pltpu
from jax.experimental.pallas import tpu_sc as plsc

N = 10000
C = 128
E = 320000
NC = 2
NS = 16
NW = NC * NS
EPT = E // NW
CH = 128
NCH = 80
EPTP = NCH * CH
NPAD = 10112
ROWS = NPAD // NS
PADROW = N

_mesh = plsc.VectorSubcoreMesh(
    core_axis_name="c", subcore_axis_name="s", num_cores=NC, num_subcores=NS)


HR = NCH

@functools.partial(
    pl.kernel,
    out_type=jax.ShapeDtypeStruct((NC, HR, CH), jnp.float32),
    mesh=_mesh,
    compiler_params=pltpu.CompilerParams(needs_layout_passes=False),
    scratch_types=[
        pltpu.VMEM((NCH, CH), jnp.int32),
        pltpu.VMEM((HR, CH), jnp.float32),
        pltpu.VMEM((HR,), jnp.int32),
        pltpu.VMEM_SHARED((HR, CH), jnp.float32),
    ],
)
def _deg_kernel(dst_hbm, rowidx_hbm, zeros_hbm, out_hbm,
                di_all, hist, rowidx_v, acc):
    cid = lax.axis_index("c")
    sid = lax.axis_index("s")
    tid = sid * NC + cid
    pltpu.sync_copy(dst_hbm.at[tid], di_all)
    pltpu.sync_copy(rowidx_hbm, rowidx_v)
    pltpu.sync_copy(zeros_hbm, acc.at[pl.ds(sid * (HR // NS), HR // NS)])

    zero16 = jnp.zeros((16,), jnp.float32)

    def zbody(r, carry):
        for k in range(8):
            hist[r, pl.ds(k * 16, 16)] = zero16
        return carry

    lax.fori_loop(0, HR, zbody, 0)
    plsc.subcore_barrier()

    ones = jnp.full((16,), 1.0, jnp.float32)

    def body(r, carry):
        for k in range(8):
            idx = di_all[r, pl.ds(k * 16, 16)]
            plsc.addupdate_scatter(hist, [idx >> 7, idx & 127], ones)
        return carry

    lax.fori_loop(0, NCH, body, 0)

    pltpu.sync_copy(hist, acc.at[rowidx_v], add=True)
    plsc.subcore_barrier()

    @pl.when(sid < HR // 8)
    def _():
        pltpu.sync_copy(acc.at[pl.ds(sid * 8, 8)],
                        out_hbm.at[cid, pl.ds(sid * 8, 8)])


SCH = 80
SNCH = EPT // SCH

@functools.partial(
    pl.kernel,
    out_type=jax.ShapeDtypeStruct((NC, NPAD, C), jnp.float32),
    mesh=_mesh,
    scratch_types=[
        pltpu.VMEM((EPT,), jnp.int32),
        pltpu.VMEM((SCH,), jnp.int32),
        pltpu.VMEM((SCH, C), jnp.float32),
        pltpu.VMEM((SCH, C), jnp.float32),
        pltpu.VMEM((SCH, C), jnp.float32),
        pltpu.SemaphoreType.DMA,
        pltpu.SemaphoreType.DMA,
        pltpu.SemaphoreType.DMA,
        pltpu.VMEM_SHARED((NPAD, C), jnp.float32),
    ],
)
def _scatter_kernel(src_hbm, dst_hbm, y_hbm, zeros_hbm, out_hbm,
                    si_all, di_v, rows0, rows1, rows2, sem0, sem1, sem2, acc):
    cid = lax.axis_index("c")
    sid = lax.axis_index("s")
    tid = sid * NC + cid
    base = tid * EPT
    pltpu.sync_copy(src_hbm.at[pl.ds(base, EPT)], si_all)
    pltpu.sync_copy(zeros_hbm, acc.at[pl.ds(sid * ROWS, ROWS)])
    plsc.subcore_barrier()

    rows = (rows0, rows1, rows2)
    sems = (sem0, sem1, sem2)
    pltpu.async_copy(y_hbm.at[si_all.at[pl.ds(0, SCH)]], rows0, sem0)
    pltpu.async_copy(y_hbm.at[si_all.at[pl.ds(SCH, SCH)]], rows1, sem1)

    def outer(j, carry):
        for b in range(3):
            i = j * 3 + b
            pltpu.sync_copy(dst_hbm.at[pl.ds(base + i * SCH, SCH)], di_v)
            pltpu.make_async_copy(y_hbm.at[si_all.at[pl.ds(i * SCH, SCH)]],
                                  rows[b], sems[b]).wait()
            nb = (b + 2) % 3
            pltpu.async_copy(y_hbm.at[si_all.at[pl.ds((i + 2) * SCH, SCH)]],
                             rows[nb], sems[nb])
            pltpu.sync_copy(rows[b], acc.at[di_v], add=True)
        return carry

    lax.fori_loop(0, (SNCH - 2) // 3, outer, 0)
    for i in (SNCH - 2, SNCH - 1):
        b = i % 3
        pltpu.make_async_copy(y_hbm.at[si_all.at[pl.ds(i * SCH, SCH)]],
                              rows[b], sems[b]).wait()
        pltpu.sync_copy(dst_hbm.at[pl.ds(base + i * SCH, SCH)], di_v)
        pltpu.sync_copy(rows[b], acc.at[di_v], add=True)
    plsc.subcore_barrier()
    pltpu.sync_copy(acc.at[pl.ds(sid * ROWS, ROWS)],
                    out_hbm.at[cid, pl.ds(sid * ROWS, ROWS)])


def _bnmm_body(x_ref, g_ref, be_ref, w_ref, d0_ref, d1_ref, y_ref):
    x = x_ref[...]
    mean = jnp.mean(x, axis=0, keepdims=True)
    xc = x - mean
    var = jnp.mean(xc * xc, axis=0, keepdims=True)
    xh = xc * lax.rsqrt(var + 1e-5) * g_ref[...] + be_ref[...]
    xw = jnp.dot(xh, w_ref[...], preferred_element_type=jnp.float32)
    deg = d0_ref[...] + d1_ref[...] + 1.0
    y_ref[...] = xw * lax.rsqrt(deg)


_bnmm_call = pl.pallas_call(
    _bnmm_body, out_shape=jax.ShapeDtypeStruct((N, C), jnp.float32))


def _fin_body(p_ref, y_ref, d0_ref, d1_ref, b_ref, o_ref):
    deg = d0_ref[...] + d1_ref[...] + 1.0
    dis = lax.rsqrt(deg)
    s = p_ref[0, 0:N] + p_ref[1, 0:N] + y_ref[...]
    o_ref[...] = jnp.maximum(s * dis + b_ref[...], 0.0)


_fin_call = pl.pallas_call(
    _fin_body, out_shape=jax.ShapeDtypeStruct((N, C), jnp.float32))


def kernel(x, edge_index, bn_gamma, bn_beta, W, b):
    src = edge_index[0].astype(jnp.int32).reshape(NW, EPT)
    dst = edge_index[1].astype(jnp.int32).reshape(NW, EPT)
    srcp = jnp.pad(src, ((0, 0), (0, EPTP - EPT))).reshape(NW, NCH, CH)
    dstp = jnp.pad(dst, ((0, 0), (0, EPTP - EPT)),
                   constant_values=PADROW).reshape(NW, NCH, CH)
    zeros_c = jnp.zeros((ROWS, C), jnp.float32)
    rowidx = jnp.arange(HR, dtype=jnp.int32)
    zeros_h = jnp.zeros((HR // NS, CH), jnp.float32)
    degp = _deg_kernel(dstp, rowidx, zeros_h)
    d0 = degp[0].reshape(HR * CH)[:N].reshape(N, 1)
    d1 = degp[1].reshape(HR * CH)[:N].reshape(N, 1)
    y = _bnmm_call(x, bn_gamma.reshape(1, C), bn_beta.reshape(1, C), W, d0, d1)
    p = _scatter_kernel(src.reshape(E), dst.reshape(E), y, zeros_c)
    return _fin_call(p, y, d0, d1, b.reshape(1, C))


# speedup vs baseline: 1.0285x; 1.0285x over previous
import functools

import jax
import jax.numpy as jnp
from jax import lax
from jax.experimental import pallas as pl
from jax.experimental.pallas import tpu as pltpu
from jax.experimental.pallas import tpu_sc as plsc

N = 10000
C = 128
E = 320000
NC = 2
NS = 16
NW = NC * NS
EPT = E // NW
CH = 128
NCH = 80
EPTP = NCH * CH
NPAD = 10112
ROWS = NPAD // NS
PADROW = N

_mesh = plsc.VectorSubcoreMesh(
    core_axis_name="c", subcore_axis_name="s", num_cores=NC, num_subcores=NS)


HR = NCH

@functools.partial(
    pl.kernel,
    out_type=jax.ShapeDtypeStruct((NC, HR, CH), jnp.float32),
    mesh=_mesh,
    compiler_params=pltpu.CompilerParams(needs_layout_passes=False),
    scratch_types=[
        pltpu.VMEM((EPT,), jnp.int32),
        pltpu.VMEM((HR, CH), jnp.float32),
        pltpu.VMEM((HR,), jnp.int32),
        pltpu.VMEM_SHARED((HR, CH), jnp.float32),
    ],
)
def _deg_kernel(dst_hbm, rowidx_hbm, zeros_hbm, out_hbm,
                di_all, hist, rowidx_v, acc):
    cid = lax.axis_index("c")
    sid = lax.axis_index("s")
    tid = sid * NC + cid
    pltpu.sync_copy(dst_hbm.at[pl.ds(tid * EPT, EPT)], di_all)
    pltpu.sync_copy(rowidx_hbm, rowidx_v)
    pltpu.sync_copy(zeros_hbm, acc.at[pl.ds(sid * (HR // NS), HR // NS)])

    zero16 = jnp.zeros((16,), jnp.float32)

    def zbody(r, carry):
        for k in range(8):
            hist[r, pl.ds(k * 16, 16)] = zero16
        return carry

    lax.fori_loop(0, HR, zbody, 0)
    plsc.subcore_barrier()

    ones = jnp.full((16,), 1.0, jnp.float32)

    def body(r, carry):
        for k in range(8):
            idx = di_all[pl.ds(r * CH + k * 16, 16)]
            plsc.addupdate_scatter(hist, [idx >> 7, idx & 127], ones)
        return carry

    lax.fori_loop(0, EPT // CH, body, 0)
    for k in range((EPT % CH) // 16):
        idx = di_all[pl.ds((EPT // CH) * CH + k * 16, 16)]
        plsc.addupdate_scatter(hist, [idx >> 7, idx & 127], ones)

    pltpu.sync_copy(hist, acc.at[rowidx_v], add=True)
    plsc.subcore_barrier()

    @pl.when(sid < HR // 8)
    def _():
        pltpu.sync_copy(acc.at[pl.ds(sid * 8, 8)],
                        out_hbm.at[cid, pl.ds(sid * 8, 8)])


SCH = 80
SNCH = EPT // SCH

@functools.partial(
    pl.kernel,
    out_type=jax.ShapeDtypeStruct((NC, NPAD, C), jnp.float32),
    mesh=_mesh,
    scratch_types=[
        pltpu.VMEM((EPT,), jnp.int32),
        pltpu.VMEM((SCH,), jnp.int32),
        pltpu.VMEM((SCH, C), jnp.float32),
        pltpu.VMEM((SCH, C), jnp.float32),
        pltpu.VMEM((SCH, C), jnp.float32),
        pltpu.SemaphoreType.DMA,
        pltpu.SemaphoreType.DMA,
        pltpu.SemaphoreType.DMA,
        pltpu.VMEM_SHARED((NPAD, C), jnp.float32),
    ],
)
def _scatter_kernel(src_hbm, dst_hbm, y_hbm, zeros_hbm, out_hbm,
                    si_all, di_v, rows0, rows1, rows2, sem0, sem1, sem2, acc):
    cid = lax.axis_index("c")
    sid = lax.axis_index("s")
    tid = sid * NC + cid
    base = tid * EPT
    pltpu.sync_copy(src_hbm.at[pl.ds(base, EPT)], si_all)
    pltpu.sync_copy(zeros_hbm, acc.at[pl.ds(sid * ROWS, ROWS)])
    plsc.subcore_barrier()

    rows = (rows0, rows1, rows2)
    sems = (sem0, sem1, sem2)
    pltpu.async_copy(y_hbm.at[si_all.at[pl.ds(0, SCH)]], rows0, sem0)
    pltpu.async_copy(y_hbm.at[si_all.at[pl.ds(SCH, SCH)]], rows1, sem1)

    def outer(j, carry):
        for b in range(3):
            i = j * 3 + b
            pltpu.make_async_copy(y_hbm.at[si_all.at[pl.ds(i * SCH, SCH)]],
                                  rows[b], sems[b]).wait()
            nb = (b + 2) % 3
            pltpu.async_copy(y_hbm.at[si_all.at[pl.ds((i + 2) * SCH, SCH)]],
                             rows[nb], sems[nb])
            pltpu.sync_copy(dst_hbm.at[pl.ds(base + i * SCH, SCH)], di_v)
            pltpu.sync_copy(rows[b], acc.at[di_v], add=True)
        return carry

    lax.fori_loop(0, (SNCH - 2) // 3, outer, 0)
    for i in (SNCH - 2, SNCH - 1):
        b = i % 3
        pltpu.make_async_copy(y_hbm.at[si_all.at[pl.ds(i * SCH, SCH)]],
                              rows[b], sems[b]).wait()
        pltpu.sync_copy(dst_hbm.at[pl.ds(base + i * SCH, SCH)], di_v)
        pltpu.sync_copy(rows[b], acc.at[di_v], add=True)
    plsc.subcore_barrier()
    pltpu.sync_copy(acc.at[pl.ds(sid * ROWS, ROWS)],
                    out_hbm.at[cid, pl.ds(sid * ROWS, ROWS)])


def _bnmm_body(x_ref, g_ref, be_ref, w_ref, d0_ref, d1_ref, y_ref):
    x = x_ref[...]
    mean = jnp.mean(x, axis=0, keepdims=True)
    xc = x - mean
    var = jnp.mean(xc * xc, axis=0, keepdims=True)
    xh = xc * lax.rsqrt(var + 1e-5) * g_ref[...] + be_ref[...]
    xw = jnp.dot(xh, w_ref[...], preferred_element_type=jnp.float32)
    deg = d0_ref[...] + d1_ref[...] + 1.0
    y_ref[...] = xw * lax.rsqrt(deg)


_bnmm_call = pl.pallas_call(
    _bnmm_body, out_shape=jax.ShapeDtypeStruct((N, C), jnp.float32))


def _fin_body(p_ref, y_ref, d0_ref, d1_ref, b_ref, o_ref):
    deg = d0_ref[...] + d1_ref[...] + 1.0
    dis = lax.rsqrt(deg)
    s = p_ref[0, 0:N] + p_ref[1, 0:N] + y_ref[...]
    o_ref[...] = jnp.maximum(s * dis + b_ref[...], 0.0)


_fin_call = pl.pallas_call(
    _fin_body, out_shape=jax.ShapeDtypeStruct((N, C), jnp.float32))


def kernel(x, edge_index, bn_gamma, bn_beta, W, b):
    src = edge_index[0].astype(jnp.int32).reshape(E)
    dst = edge_index[1].astype(jnp.int32).reshape(E)
    zeros_c = jnp.zeros((ROWS, C), jnp.float32)
    rowidx = jnp.arange(HR, dtype=jnp.int32)
    zeros_h = jnp.zeros((HR // NS, CH), jnp.float32)
    degp = _deg_kernel(dst, rowidx, zeros_h)
    d0 = degp[0].reshape(HR * CH)[:N].reshape(N, 1)
    d1 = degp[1].reshape(HR * CH)[:N].reshape(N, 1)
    y = _bnmm_call(x, bn_gamma.reshape(1, C), bn_beta.reshape(1, C), W, d0, d1)
    p = _scatter_kernel(src, dst, y, zeros_c)
    return _fin_call(p, y, d0, d1, b.reshape(1, C))
